# baseline (speedup 1.0000x reference)
import functools
import math

import jax
import jax.numpy as jnp
from jax import lax
from jax.experimental import pallas as pl
from jax.experimental.pallas import tpu as pltpu
from jax.experimental.pallas import tpu_sc as plsc

B, T, D, E, F = 2, 4096, 1024, 16, 4096
N = B * T
CAP = N // E
LANES = 16
FJ = 4
FC = F // FJ


def _cumsum_lanes(x, l128, l64s):
    xb = x.reshape(E, N // 128, 128)
    within = lax.dot_general(
        xb, l128, (((2,), (0,)), ((), ())),
        preferred_element_type=jnp.float32,
    )
    bs = jnp.sum(xb, axis=2)
    bo = lax.dot_general(
        bs, l64s, (((1,), (0,)), ((), ())),
        preferred_element_type=jnp.float32,
    )
    return (within + bo[:, :, None]).reshape(E, N)


def _logits_body(x_ref, w_ref, logits_ref):
    xb = x_ref[...]
    logits_ref[...] = lax.dot_general(
        xb, w_ref[...], (((1,), (0,)), ((), ())),
        preferred_element_type=jnp.float32,
    )


_RB = 1024


def _logits(x_flat, w_router):
    return pl.pallas_call(
        _logits_body,
        grid=(N // _RB,),
        in_specs=[
            pl.BlockSpec((_RB, D), lambda i: (i, 0)),
            pl.BlockSpec((D, E), lambda i: (0, 0)),
        ],
        out_specs=pl.BlockSpec((_RB, E), lambda i: (i, 0)),
        out_shape=jax.ShapeDtypeStruct((N, E), jnp.float32),
    )(x_flat, w_router)


def _router_body(logits_ref, l128_ref, l64s_ref, scores_ref, posn_ref):
    logits = logits_ref[...]
    m = jnp.max(logits, axis=0, keepdims=True)
    p = jnp.exp(logits - m)
    s = jnp.sum(p, axis=0, keepdims=True)
    scores_ne = p / s
    scores = scores_ne.T
    scores_ref[...] = scores

    u = lax.bitcast_convert_type(scores, jnp.int32)

    def bit_step(i, prefix):
        bit = 30 - i
        trial = prefix | lax.shift_left(jnp.int32(1), bit)
        cnt = jnp.sum((u >= trial).astype(jnp.float32), axis=1, keepdims=True)
        return jnp.where(cnt >= float(CAP), trial, prefix)

    prefix = lax.fori_loop(0, 31, bit_step, jnp.zeros((E, 1), jnp.int32))
    thr = lax.bitcast_convert_type(prefix, jnp.float32)

    l128 = l128_ref[...]
    l64s = l64s_ref[...]
    gt = (scores > thr).astype(jnp.float32)
    eq = (scores == thr).astype(jnp.float32)
    c_gt = jnp.sum(gt, axis=1, keepdims=True)
    quota = float(CAP) - c_gt
    eq_rank = _cumsum_lanes(eq, l128, l64s)
    sel = gt + eq * (eq_rank <= quota).astype(jnp.float32)
    pos = _cumsum_lanes(sel, l128, l64s) - 1.0
    posn = jnp.where(sel > 0.0, pos, -1.0).astype(jnp.int32)
    posn_ref[...] = posn


def _router(logits):
    nb = N // 128
    l128 = (jnp.arange(128)[:, None] <= jnp.arange(128)[None, :]).astype(
        jnp.float32)
    l64s = (jnp.arange(nb)[:, None] < jnp.arange(nb)[None, :]).astype(
        jnp.float32)
    return pl.pallas_call(
        _router_body,
        out_shape=(
            jax.ShapeDtypeStruct((E, N), jnp.float32),
            jax.ShapeDtypeStruct((E, N), jnp.int32),
        ),
    )(logits, l128, l64s)


def _compact_body(scores_hbm, posn_hbm, idx_hbm, wts_hbm,
                  score_v, pos_v, idx_v, wts_v):
    cid = lax.axis_index("c")
    sid = lax.axis_index("s")
    wid = sid * 2 + cid

    @pl.when(wid < E)
    def _():
        pltpu.sync_copy(scores_hbm.at[wid], score_v)
        pltpu.sync_copy(posn_hbm.at[wid], pos_v)

        def sel_body(i, _):
            p = pos_v[pl.ds(i * LANES, LANES)]
            m = p >= 0
            tok = lax.iota(jnp.int32, LANES) + i * LANES
            v = score_v[pl.ds(i * LANES, LANES)]
            plsc.store_scatter(idx_v, [p], tok, mask=m)
            plsc.store_scatter(wts_v, [p], v, mask=m)
            return 0

        lax.fori_loop(0, N // LANES, sel_body, 0)
        pltpu.sync_copy(idx_v, idx_hbm.at[wid])
        pltpu.sync_copy(wts_v, wts_hbm.at[wid])


def _compact(scores, posn):
    mesh = plsc.VectorSubcoreMesh(core_axis_name="c", subcore_axis_name="s")
    kern = pl.kernel(
        _compact_body,
        out_type=(
            jax.ShapeDtypeStruct((E, CAP), jnp.int32),
            jax.ShapeDtypeStruct((E, CAP), jnp.float32),
        ),
        mesh=mesh,
        scratch_types=[
            pltpu.VMEM((N,), jnp.float32),
            pltpu.VMEM((N,), jnp.int32),
            pltpu.VMEM((CAP,), jnp.int32),
            pltpu.VMEM((CAP,), jnp.float32),
        ],
        compiler_params=pltpu.CompilerParams(needs_layout_passes=False),
    )
    return kern(scores, posn)


_G_ROWS = N // 32
_G_CHUNK = 32


def _gather_body(x_hbm, idx_hbm, disp_hbm, idx_v, buf0, buf1, sem0, sem1):
    cid = lax.axis_index("c")
    sid = lax.axis_index("s")
    wid = sid * 2 + cid
    base = wid * _G_ROWS
    pltpu.sync_copy(idx_hbm.at[pl.ds(base, _G_ROWS)], idx_v)

    bufs = (buf0, buf1)
    sems = (sem0, sem1)
    nch = _G_ROWS // _G_CHUNK

    def start(c, b):
        return pltpu.async_copy(
            x_hbm.at[idx_v.at[pl.ds(c * _G_CHUNK, _G_CHUNK)]], bufs[b],
            sems[b])

    cp = start(0, 0)
    for c in range(nch):
        b = c % 2
        cp.wait()
        if c + 1 < nch:
            cp = start(c + 1, 1 - b)
        pltpu.sync_copy(bufs[b],
                        disp_hbm.at[pl.ds(base + c * _G_CHUNK, _G_CHUNK)])


def _gather(x_flat, idx_flat):
    mesh = plsc.VectorSubcoreMesh(core_axis_name="c", subcore_axis_name="s")
    kern = pl.kernel(
        _gather_body,
        out_type=jax.ShapeDtypeStruct((N, D), jnp.float32),
        mesh=mesh,
        scratch_types=[
            pltpu.VMEM((_G_ROWS,), jnp.int32),
            pltpu.VMEM((_G_CHUNK, D), jnp.float32),
            pltpu.VMEM((_G_CHUNK, D), jnp.float32),
            pltpu.SemaphoreType.DMA,
            pltpu.SemaphoreType.DMA,
        ],
    )
    return kern(x_flat, idx_flat)


def _ffn_body(idx_smem, disp_ref, f1w_ref, f1b_ref, f2w_ref, f2b_ref, wts_ref,
              out_ref, acc_ref):
    e = pl.program_id(0)
    fj = pl.program_id(1)

    xv = disp_ref[0].astype(jnp.bfloat16)
    h = jnp.dot(xv, f1w_ref[0].astype(jnp.bfloat16),
                preferred_element_type=jnp.float32)
    h = h + f1b_ref[0, 0][None, :]
    h = 0.5 * h * (1.0 + lax.erf(h * (1.0 / math.sqrt(2.0))))
    part = jnp.dot(h.astype(jnp.bfloat16), f2w_ref[0].astype(jnp.bfloat16),
                   preferred_element_type=jnp.float32)
    part3 = part.reshape(CAP, 8, 128)

    @pl.when(fj == 0)
    def _():
        bias3 = f2b_ref[0, 0].reshape(1, 8, 128)
        acc_ref[...] = part3 + bias3

    @pl.when(fj != 0)
    def _():
        acc_ref[...] = acc_ref[...] + part3

    @pl.when(fj == FJ - 1)
    def _():
        @pl.when(e == 0)
        def _():
            out_ref[...] = jnp.zeros_like(out_ref)

        w = wts_ref[0, 0].reshape(CAP, 1, 1)
        acc_ref[...] = acc_ref[...] * w

        def row_body(c, _):
            tok = idx_smem[e * CAP + c]
            out_ref[tok] = out_ref[tok] + acc_ref[c]
            return 0

        lax.fori_loop(0, CAP, row_body, 0, unroll=16)


def _ffn_scatter(disp, fc1_w, fc1_b, fc2_w, fc2_b, idx_flat, wts):
    disp4 = disp.reshape(E, CAP, D)
    grid = (E, FJ)
    out = pl.pallas_call(
        _ffn_body,
        grid=grid,
        in_specs=[
            pl.BlockSpec(memory_space=pltpu.SMEM),
            pl.BlockSpec((1, CAP, D), lambda e, fj: (e, 0, 0)),
            pl.BlockSpec((1, D, FC), lambda e, fj: (e, 0, fj)),
            pl.BlockSpec((1, 1, FC), lambda e, fj: (e, 0, fj)),
            pl.BlockSpec((1, FC, D), lambda e, fj: (e, fj, 0)),
            pl.BlockSpec((1, 1, D), lambda e, fj: (e, 0, 0)),
            pl.BlockSpec((1, 1, CAP), lambda e, fj: (e, 0, 0)),
        ],
        out_specs=pl.BlockSpec((N, 8, 128), lambda e, fj: (0, 0, 0)),
        out_shape=jax.ShapeDtypeStruct((N, 8, 128), jnp.float32),
        scratch_shapes=[pltpu.VMEM((CAP, 8, 128), jnp.float32)],
        compiler_params=pltpu.CompilerParams(
            vmem_limit_bytes=64 * 1024 * 1024,
        ),
    )(idx_flat, disp4, fc1_w, fc1_b.reshape(E, 1, F), fc2_w,
      fc2_b.reshape(E, 1, D), wts.reshape(E, 1, CAP))
    return out


def kernel(x, W_router, fc1_w, fc1_b, fc2_w, fc2_b):
    x_flat = x.reshape(N, D)
    logits = _logits(x_flat, W_router)
    scores, posn = _router(logits)
    idx, wts = _compact(scores, posn)
    idx_flat = idx.reshape(-1)
    disp = _gather(x_flat, idx_flat)
    out = _ffn_scatter(disp, fc1_w, fc1_b, fc2_w, fc2_b, idx_flat, wts)
    output = out.reshape(B, T, D)
    frac = jnp.float32(CAP) / jnp.float32(N)
    aux_loss = jnp.float32(frac * frac)
    mean_cap_util = jnp.float32(frac)
    return (output, aux_loss, mean_cap_util)


# speedup vs baseline: 1.0024x; 1.0024x over previous
import math

import jax
import jax.numpy as jnp
from jax import lax
from jax.experimental import pallas as pl
from jax.experimental.pallas import tpu as pltpu
from jax.experimental.pallas import tpu_sc as plsc

B, T, D, E, F = 2, 4096, 1024, 16, 4096
N = B * T
CAP = N // E
LANES = 16
FJ = 4
FC = F // FJ


def _cumsum_lanes(x, l128, l64s):
    xb = x.reshape(E, N // 128, 128)
    within = lax.dot_general(
        xb, l128, (((2,), (0,)), ((), ())),
        preferred_element_type=jnp.float32,
    )
    bs = jnp.sum(xb, axis=2)
    bo = lax.dot_general(
        bs, l64s, (((1,), (0,)), ((), ())),
        preferred_element_type=jnp.float32,
    )
    return (within + bo[:, :, None]).reshape(E, N)


def _logits_body(x_ref, w_ref, logits_ref):
    xb = x_ref[...]
    logits_ref[...] = lax.dot_general(
        xb, w_ref[...], (((1,), (0,)), ((), ())),
        preferred_element_type=jnp.float32,
    )


_RB = 1024


def _logits(x_flat, w_router):
    return pl.pallas_call(
        _logits_body,
        grid=(N // _RB,),
        in_specs=[
            pl.BlockSpec((_RB, D), lambda i: (i, 0)),
            pl.BlockSpec((D, E), lambda i: (0, 0)),
        ],
        out_specs=pl.BlockSpec((_RB, E), lambda i: (i, 0)),
        out_shape=jax.ShapeDtypeStruct((N, E), jnp.float32),
    )(x_flat, w_router)


def _router_body(logits_ref, l128_ref, l64s_ref, scores_ref, posn_ref):
    logits = logits_ref[...]
    m = jnp.max(logits, axis=0, keepdims=True)
    p = jnp.exp(logits - m)
    s = jnp.sum(p, axis=0, keepdims=True)
    scores_ne = p / s
    scores = scores_ne.T
    scores_ref[...] = scores

    u = lax.bitcast_convert_type(scores, jnp.int32)

    def bit_step(i, prefix):
        bit = 30 - i
        trial = prefix | lax.shift_left(jnp.int32(1), bit)
        cnt = jnp.sum((u >= trial).astype(jnp.float32), axis=1, keepdims=True)
        return jnp.where(cnt >= float(CAP), trial, prefix)

    prefix = lax.fori_loop(0, 31, bit_step, jnp.zeros((E, 1), jnp.int32))
    thr = lax.bitcast_convert_type(prefix, jnp.float32)

    l128 = l128_ref[...]
    l64s = l64s_ref[...]
    gt = (scores > thr).astype(jnp.float32)
    eq = (scores == thr).astype(jnp.float32)
    c_gt = jnp.sum(gt, axis=1, keepdims=True)
    quota = float(CAP) - c_gt
    eq_rank = _cumsum_lanes(eq, l128, l64s)
    sel = gt + eq * (eq_rank <= quota).astype(jnp.float32)
    pos = _cumsum_lanes(sel, l128, l64s) - 1.0
    posn = jnp.where(sel > 0.0, pos, -1.0).astype(jnp.int32)
    posn_ref[...] = posn


def _router(logits):
    nb = N // 128
    l128 = (jnp.arange(128)[:, None] <= jnp.arange(128)[None, :]).astype(
        jnp.float32)
    l64s = (jnp.arange(nb)[:, None] < jnp.arange(nb)[None, :]).astype(
        jnp.float32)
    return pl.pallas_call(
        _router_body,
        out_shape=(
            jax.ShapeDtypeStruct((E, N), jnp.float32),
            jax.ShapeDtypeStruct((E, N), jnp.int32),
        ),
    )(logits, l128, l64s)


def _compact_body(scores_hbm, posn_hbm, idx_hbm, wts_hbm,
                  score_v, pos_v, idx_v, wts_v):
    cid = lax.axis_index("c")
    sid = lax.axis_index("s")
    wid = sid * 2 + cid

    @pl.when(wid < E)
    def _():
        pltpu.sync_copy(scores_hbm.at[wid], score_v)
        pltpu.sync_copy(posn_hbm.at[wid], pos_v)

        def sel_body(i, _):
            p = pos_v[pl.ds(i * LANES, LANES)]
            m = p >= 0
            tok = lax.iota(jnp.int32, LANES) + i * LANES
            v = score_v[pl.ds(i * LANES, LANES)]
            plsc.store_scatter(idx_v, [p], tok, mask=m)
            plsc.store_scatter(wts_v, [p], v, mask=m)
            return 0

        lax.fori_loop(0, N // LANES, sel_body, 0)
        pltpu.sync_copy(idx_v, idx_hbm.at[wid])
        pltpu.sync_copy(wts_v, wts_hbm.at[wid])


def _compact(scores, posn):
    mesh = plsc.VectorSubcoreMesh(core_axis_name="c", subcore_axis_name="s")
    kern = pl.kernel(
        _compact_body,
        out_type=(
            jax.ShapeDtypeStruct((E, CAP), jnp.int32),
            jax.ShapeDtypeStruct((E, CAP), jnp.float32),
        ),
        mesh=mesh,
        scratch_types=[
            pltpu.VMEM((N,), jnp.float32),
            pltpu.VMEM((N,), jnp.int32),
            pltpu.VMEM((CAP,), jnp.int32),
            pltpu.VMEM((CAP,), jnp.float32),
        ],
        compiler_params=pltpu.CompilerParams(needs_layout_passes=False),
    )
    return kern(scores, posn)


_G_ROWS = N // 32
_G_CHUNK = 32


def _gather_body(x_hbm, idx_hbm, disp_hbm, idx_v, buf0, buf1, sem0, sem1):
    cid = lax.axis_index("c")
    sid = lax.axis_index("s")
    wid = sid * 2 + cid
    base = wid * _G_ROWS
    pltpu.sync_copy(idx_hbm.at[pl.ds(base, _G_ROWS)], idx_v)

    bufs = (buf0, buf1)
    sems = (sem0, sem1)
    nch = _G_ROWS // _G_CHUNK

    def start(c, b):
        return pltpu.async_copy(
            x_hbm.at[idx_v.at[pl.ds(c * _G_CHUNK, _G_CHUNK)]], bufs[b],
            sems[b])

    cp = start(0, 0)
    for c in range(nch):
        b = c % 2
        cp.wait()
        if c + 1 < nch:
            cp = start(c + 1, 1 - b)
        pltpu.sync_copy(bufs[b],
                        disp_hbm.at[pl.ds(base + c * _G_CHUNK, _G_CHUNK)])


def _gather(x_flat, idx_flat):
    mesh = plsc.VectorSubcoreMesh(core_axis_name="c", subcore_axis_name="s")
    kern = pl.kernel(
        _gather_body,
        out_type=jax.ShapeDtypeStruct((N, D), jnp.float32),
        mesh=mesh,
        scratch_types=[
            pltpu.VMEM((_G_ROWS,), jnp.int32),
            pltpu.VMEM((_G_CHUNK, D), jnp.float32),
            pltpu.VMEM((_G_CHUNK, D), jnp.float32),
            pltpu.SemaphoreType.DMA,
            pltpu.SemaphoreType.DMA,
        ],
    )
    return kern(x_flat, idx_flat)


def _ffn_body(idx_smem, disp_ref, f1w_ref, f1b_ref, f2w_ref, f2b_ref, wts_ref,
              out_ref, acc_ref):
    e = pl.program_id(0)
    fj = pl.program_id(1)

    xv = disp_ref[0].astype(jnp.bfloat16)
    h = jnp.dot(xv, f1w_ref[0].astype(jnp.bfloat16),
                preferred_element_type=jnp.float32)
    h = h + f1b_ref[0, 0][None, :]
    h = 0.5 * h * (1.0 + lax.erf(h * (1.0 / math.sqrt(2.0))))
    part = jnp.dot(h.astype(jnp.bfloat16), f2w_ref[0].astype(jnp.bfloat16),
                   preferred_element_type=jnp.float32)
    part3 = part.reshape(CAP, 8, 128)

    @pl.when(fj == 0)
    def _():
        bias3 = f2b_ref[0, 0].reshape(1, 8, 128)
        acc_ref[...] = part3 + bias3

    @pl.when(fj != 0)
    def _():
        acc_ref[...] = acc_ref[...] + part3

    @pl.when(fj == FJ - 1)
    def _():
        @pl.when(e == 0)
        def _():
            out_ref[...] = jnp.zeros_like(out_ref)

        w = wts_ref[0, 0].reshape(CAP, 1, 1)
        acc_ref[...] = acc_ref[...] * w

        def row_body(c, _):
            tok = idx_smem[e * CAP + c]
            out_ref[tok] = out_ref[tok] + acc_ref[c]
            return 0

        lax.fori_loop(0, CAP, row_body, 0, unroll=16)


def _ffn_scatter(disp, fc1_w, fc1_b, fc2_w, fc2_b, idx_flat, wts):
    disp4 = disp.reshape(E, CAP, D)
    grid = (E, FJ)
    out = pl.pallas_call(
        _ffn_body,
        grid=grid,
        in_specs=[
            pl.BlockSpec(memory_space=pltpu.SMEM),
            pl.BlockSpec((1, CAP, D), lambda e, fj: (e, 0, 0)),
            pl.BlockSpec((1, D, FC), lambda e, fj: (e, 0, fj)),
            pl.BlockSpec((1, 1, FC), lambda e, fj: (e, 0, fj)),
            pl.BlockSpec((1, FC, D), lambda e, fj: (e, fj, 0)),
            pl.BlockSpec((1, 1, D), lambda e, fj: (e, 0, 0)),
            pl.BlockSpec((1, 1, CAP), lambda e, fj: (e, 0, 0)),
        ],
        out_specs=pl.BlockSpec((N, 8, 128), lambda e, fj: (0, 0, 0)),
        out_shape=jax.ShapeDtypeStruct((N, 8, 128), jnp.float32),
        scratch_shapes=[pltpu.VMEM((CAP, 8, 128), jnp.float32)],
        compiler_params=pltpu.CompilerParams(
            vmem_limit_bytes=64 * 1024 * 1024,
        ),
    )(idx_flat, disp4, fc1_w, fc1_b.reshape(E, 1, F), fc2_w,
      fc2_b.reshape(E, 1, D), wts.reshape(E, 1, CAP))
    return out


def kernel(x, W_router, fc1_w, fc1_b, fc2_w, fc2_b):
    x_flat = x.reshape(N, D)
    logits = _logits(x_flat, W_router)
    scores, posn = _router(logits)
    idx, wts = _compact(scores, posn)
    idx_flat = idx.reshape(-1)
    disp = _gather(x_flat, idx_flat)
    out = _ffn_scatter(disp, fc1_w, fc1_b, fc2_w, fc2_b, idx_flat, wts)
    output = out.reshape(B, T, D)
    frac = jnp.float32(CAP) / jnp.float32(N)
    aux_loss = jnp.float32(frac * frac)
    mean_cap_util = jnp.float32(frac)
    return (output, aux_loss, mean_cap_util)
